# Initial kernel scaffold; baseline (speedup 1.0000x reference)
import jax
import jax.numpy as jnp
from jax.experimental import pallas as pl


def kernel(x, edge_index, W1, W2):
    raise NotImplementedError("write your pallas kernel here")


# speedup vs baseline: 12.7030x; 12.7030x over previous
import functools

import jax
import jax.numpy as jnp
from jax import lax
from jax.experimental import pallas as pl
from jax.experimental.pallas import tpu as pltpu
from jax.experimental.pallas import tpu_sc as plsc

N = 10000
E = 320000
D = 128
H = 32
CP = 8

NW = 32
EPW = E // NW
CK = 125
CHUNKS = EPW // CK
NP = 10240
RPT = NP // 16


def _make_edge_pass(feat):
    mesh = plsc.VectorSubcoreMesh(core_axis_name="c", subcore_axis_name="s")

    @functools.partial(
        pl.kernel,
        out_type=jax.ShapeDtypeStruct((2, NP, feat), jnp.float32),
        mesh=mesh,
        compiler_params=pltpu.CompilerParams(use_tc_tiling_on_sc=False),
        scratch_types=[
            pltpu.VMEM((CHUNKS, CK), jnp.int32),
            pltpu.VMEM((CHUNKS, CK), jnp.int32),
            pltpu.VMEM((CK, feat), jnp.float32),
            pltpu.VMEM_SHARED((NP, feat), jnp.float32),
        ],
    )
    def edge_pass(rows_hbm, src_hbm, dst_hbm, zeros_hbm, out_hbm,
                  src_v, dst_v, rows_v, acc):
        cid = lax.axis_index("c")
        sid = lax.axis_index("s")
        wid = sid * 2 + cid
        r0 = sid * RPT
        pltpu.sync_copy(zeros_hbm.at[pl.ds(r0, RPT)], acc.at[pl.ds(r0, RPT)])
        pltpu.sync_copy(src_hbm.at[wid], src_v)
        pltpu.sync_copy(dst_hbm.at[wid], dst_v)
        plsc.subcore_barrier()

        def body(j, carry):
            pltpu.sync_copy(rows_hbm.at[src_v.at[j]], rows_v)
            pltpu.sync_copy(rows_v, acc.at[dst_v.at[j]], add=True)
            return carry

        lax.fori_loop(0, CHUNKS, body, 0)
        plsc.subcore_barrier()
        pltpu.sync_copy(acc.at[pl.ds(r0, RPT)],
                        out_hbm.at[cid, pl.ds(r0, RPT)])

    return edge_pass


_edge_pass_h = _make_edge_pass(H)
_edge_pass_c = _make_edge_pass(CP)


def _matmul1(x, W1):
    def body(x_ref, w_ref, o_ref):
        o_ref[...] = jnp.dot(x_ref[...], w_ref[...],
                             preferred_element_type=jnp.float32)

    return pl.pallas_call(
        body,
        grid=(10,),
        in_specs=[pl.BlockSpec((N // 10, D), lambda i: (i, 0)),
                  pl.BlockSpec((D, H), lambda i: (0, 0))],
        out_specs=pl.BlockSpec((N // 10, H), lambda i: (i, 0)),
        out_shape=jax.ShapeDtypeStruct((N, H), jnp.float32),
    )(x, W1)


def _layer2(p, W2p):
    def body(p_ref, w_ref, o_ref):
        h = jnp.maximum(p_ref[0] + p_ref[1], 0.0)
        o_ref[...] = jnp.dot(h, w_ref[...], preferred_element_type=jnp.float32)

    return pl.pallas_call(
        body,
        grid=(10,),
        in_specs=[pl.BlockSpec((2, NP // 10, H), lambda i: (0, i, 0)),
                  pl.BlockSpec((H, CP), lambda i: (0, 0))],
        out_specs=pl.BlockSpec((NP // 10, CP), lambda i: (i, 0)),
        out_shape=jax.ShapeDtypeStruct((NP, CP), jnp.float32),
    )(p, W2p)


def _softmax2(p):
    def body(p_ref, o_ref):
        a = p_ref[0] + p_ref[1]
        col = lax.broadcasted_iota(jnp.int32, a.shape, 1)
        logits = jnp.where(col < 2, a, -1e30)
        m = jnp.max(logits, axis=1, keepdims=True)
        e = jnp.exp(logits - m)
        o_ref[...] = e / jnp.sum(e, axis=1, keepdims=True)

    return pl.pallas_call(
        body,
        grid=(10,),
        in_specs=[pl.BlockSpec((2, NP // 10, CP), lambda i: (0, i, 0))],
        out_specs=pl.BlockSpec((NP // 10, CP), lambda i: (i, 0)),
        out_shape=jax.ShapeDtypeStruct((NP, CP), jnp.float32),
    )(p)


def kernel(x, edge_index, W1, W2):
    src = edge_index[0].reshape(NW, CHUNKS, CK)
    dst = edge_index[1].reshape(NW, CHUNKS, CK)
    zeros_h = jnp.zeros((NP, H), jnp.float32)
    zeros_c = jnp.zeros((NP, CP), jnp.float32)
    W2p = jnp.pad(W2, ((0, 0), (0, CP - W2.shape[1])))

    s1 = _matmul1(x, W1)
    p1 = _edge_pass_h(s1, src, dst, zeros_h)
    s2 = _layer2(p1, W2p)
    p2 = _edge_pass_c(s2, src, dst, zeros_c)
    outp = _softmax2(p2)
    return outp[:N, :2]


# speedup vs baseline: 20.7699x; 1.6350x over previous
import functools

import jax
import jax.numpy as jnp
from jax import lax
from jax.experimental import pallas as pl
from jax.experimental.pallas import tpu as pltpu
from jax.experimental.pallas import tpu_sc as plsc

N = 10000
E = 320000
D = 128
H = 32
CP = 8

NW = 32
EPW = E // NW
CK = 125
CHUNKS = EPW // CK
NP = 10240
RPT = NP // 16


def _make_edge_pass(feat):
    mesh = plsc.VectorSubcoreMesh(core_axis_name="c", subcore_axis_name="s")

    @functools.partial(
        pl.kernel,
        out_type=jax.ShapeDtypeStruct((2, NP, feat), jnp.float32),
        mesh=mesh,
        compiler_params=pltpu.CompilerParams(use_tc_tiling_on_sc=False),
        scratch_types=[
            pltpu.VMEM((CHUNKS, CK), jnp.int32),
            pltpu.VMEM((CHUNKS, CK), jnp.int32),
            [pltpu.VMEM((CK, feat), jnp.float32) for _ in range(4)],
            [pltpu.SemaphoreType.DMA for _ in range(4)],
            pltpu.VMEM_SHARED((NP, feat), jnp.float32),
        ],
    )
    def edge_pass(rows_hbm, src_hbm, dst_hbm, zeros_hbm, out_hbm,
                  src_v, dst_v, bufs, sems, acc):
        cid = lax.axis_index("c")
        sid = lax.axis_index("s")
        wid = sid * 2 + cid
        r0 = sid * RPT
        pltpu.sync_copy(zeros_hbm.at[pl.ds(r0, RPT)], acc.at[pl.ds(r0, RPT)])
        pltpu.sync_copy(src_hbm.at[wid], src_v)
        pltpu.sync_copy(dst_hbm.at[wid], dst_v)
        plsc.subcore_barrier()

        NB = 4
        for b in range(NB):
            pltpu.async_copy(rows_hbm.at[src_v.at[b]], bufs[b], sems[b])

        def body(i, carry):
            for b in range(NB):
                j = i * NB + b
                pltpu.make_async_copy(
                    rows_hbm.at[src_v.at[j]], bufs[b], sems[b]).wait()
                pltpu.sync_copy(bufs[b], acc.at[dst_v.at[j]], add=True)

                @pl.when(j + NB < CHUNKS)
                def _():
                    pltpu.async_copy(
                        rows_hbm.at[src_v.at[j + NB]], bufs[b], sems[b])
            return carry

        lax.fori_loop(0, CHUNKS // NB, body, 0)
        plsc.subcore_barrier()
        pltpu.sync_copy(acc.at[pl.ds(r0, RPT)],
                        out_hbm.at[cid, pl.ds(r0, RPT)])

    return edge_pass


_edge_pass_h = _make_edge_pass(H)
_edge_pass_c = _make_edge_pass(CP)


def _matmul1(x, W1):
    def body(x_ref, w_ref, o_ref):
        o_ref[...] = jnp.dot(x_ref[...], w_ref[...],
                             preferred_element_type=jnp.float32)

    return pl.pallas_call(
        body,
        grid=(10,),
        in_specs=[pl.BlockSpec((N // 10, D), lambda i: (i, 0)),
                  pl.BlockSpec((D, H), lambda i: (0, 0))],
        out_specs=pl.BlockSpec((N // 10, H), lambda i: (i, 0)),
        out_shape=jax.ShapeDtypeStruct((N, H), jnp.float32),
    )(x, W1)


def _layer2(p, W2p):
    def body(p_ref, w_ref, o_ref):
        h = jnp.maximum(p_ref[0] + p_ref[1], 0.0)
        o_ref[...] = jnp.dot(h, w_ref[...], preferred_element_type=jnp.float32)

    return pl.pallas_call(
        body,
        grid=(10,),
        in_specs=[pl.BlockSpec((2, NP // 10, H), lambda i: (0, i, 0)),
                  pl.BlockSpec((H, CP), lambda i: (0, 0))],
        out_specs=pl.BlockSpec((NP // 10, CP), lambda i: (i, 0)),
        out_shape=jax.ShapeDtypeStruct((NP, CP), jnp.float32),
    )(p, W2p)


def _softmax2(p):
    def body(p_ref, o_ref):
        a = p_ref[0] + p_ref[1]
        col = lax.broadcasted_iota(jnp.int32, a.shape, 1)
        logits = jnp.where(col < 2, a, -1e30)
        m = jnp.max(logits, axis=1, keepdims=True)
        e = jnp.exp(logits - m)
        o_ref[...] = e / jnp.sum(e, axis=1, keepdims=True)

    return pl.pallas_call(
        body,
        grid=(10,),
        in_specs=[pl.BlockSpec((2, NP // 10, CP), lambda i: (0, i, 0))],
        out_specs=pl.BlockSpec((NP // 10, CP), lambda i: (i, 0)),
        out_shape=jax.ShapeDtypeStruct((NP, CP), jnp.float32),
    )(p)


def kernel(x, edge_index, W1, W2):
    src = edge_index[0].reshape(NW, CHUNKS, CK)
    dst = edge_index[1].reshape(NW, CHUNKS, CK)
    zeros_h = jnp.zeros((NP, H), jnp.float32)
    zeros_c = jnp.zeros((NP, CP), jnp.float32)
    W2p = jnp.pad(W2, ((0, 0), (0, CP - W2.shape[1])))

    s1 = _matmul1(x, W1)
    p1 = _edge_pass_h(s1, src, dst, zeros_h)
    s2 = _layer2(p1, W2p)
    p2 = _edge_pass_c(s2, src, dst, zeros_c)
    outp = _softmax2(p2)
    return outp[:N, :2]
